# Initial kernel scaffold; baseline (speedup 1.0000x reference)
import functools

import jax
import jax.numpy as jnp
from jax import lax
from jax.experimental import pallas as pl
from jax.experimental.pallas import tpu as pltpu
from jax.experimental.pallas import tpu_sc as plsc

N = 10000
E = 320000
F_IN = 128
H = 64
C = 1
G = 128

NC = 2
NS = 16
NW = NC * NS
K = 80
CH = E // NW // K
ROWS_PER_W = E // NW // K
DW = 16

_mesh = plsc.VectorSubcoreMesh(core_axis_name="c", subcore_axis_name="s")


def _deg_body(dst_hbm, zero_hbm, ones_hbm, out_hbm, acc_sh, dst_v, ones_v):
    c = lax.axis_index("c")
    s = lax.axis_index("s")

    @pl.when(s == 0)
    def _():
        pltpu.sync_copy(zero_hbm, acc_sh)

    plsc.subcore_barrier()
    pltpu.sync_copy(ones_hbm, ones_v)
    row0 = c * (E // K // NC) + s * ROWS_PER_W
    pltpu.sync_copy(dst_hbm.at[pl.ds(row0, ROWS_PER_W)], dst_v)

    def chunk(i, carry):
        pltpu.sync_copy(ones_v, acc_sh.at[dst_v.at[i]], add=True)
        return carry

    lax.fori_loop(0, CH, chunk, 0)
    plsc.subcore_barrier()

    @pl.when(s == 0)
    def _():
        pltpu.sync_copy(acc_sh, out_hbm.at[c])


_deg_call = pl.kernel(
    _deg_body,
    out_type=jax.ShapeDtypeStruct((NC, N, DW), jnp.float32),
    mesh=_mesh,
    scratch_types=[
        pltpu.VMEM_SHARED((N, DW), jnp.float32),
        pltpu.VMEM((ROWS_PER_W, K), jnp.int32),
        pltpu.VMEM((K, DW), jnp.float32),
    ],
)


def _scatter_body(y_hbm, src_hbm, dst_hbm, zero_hbm, out_hbm,
                  y_sh, acc_sh, src_v, dst_v, rows_v, sem):
    c = lax.axis_index("c")
    s = lax.axis_index("s")

    @pl.when(s == 0)
    def _():
        pltpu.sync_copy(y_hbm, y_sh)
        pltpu.sync_copy(zero_hbm, acc_sh)

    plsc.subcore_barrier()
    row0 = c * (E // K // NC) + s * ROWS_PER_W
    pltpu.sync_copy(src_hbm.at[pl.ds(row0, ROWS_PER_W)], src_v)
    pltpu.sync_copy(dst_hbm.at[pl.ds(row0, ROWS_PER_W)], dst_v)

    def chunk(i, carry):
        pltpu.async_copy(y_sh.at[src_v.at[i]], rows_v, sem).wait()
        pltpu.sync_copy(rows_v, acc_sh.at[dst_v.at[i]], add=True)
        return carry

    lax.fori_loop(0, CH, chunk, 0)
    plsc.subcore_barrier()

    @pl.when(s == 0)
    def _():
        pltpu.sync_copy(acc_sh, out_hbm.at[c])


_scatter_call = pl.kernel(
    _scatter_body,
    out_type=jax.ShapeDtypeStruct((NC, N, H), jnp.float32),
    mesh=_mesh,
    scratch_types=[
        pltpu.VMEM_SHARED((N, H), jnp.float32),
        pltpu.VMEM_SHARED((N, H), jnp.float32),
        pltpu.VMEM((ROWS_PER_W, K), jnp.int32),
        pltpu.VMEM((ROWS_PER_W, K), jnp.int32),
        pltpu.VMEM((K, H), jnp.float32),
        pltpu.SemaphoreType.DMA,
    ],
)

SEG_ROWS = 312
SEG_LAST = N - 31 * SEG_ROWS
_LANE16 = 16


def _segmax_body(h_hbm, batch_hbm, zero_hbm, out_hbm, h_v, batch_v, slab_v):
    c = lax.axis_index("c")
    s = lax.axis_index("s")
    wid = s * NC + c
    lo = wid * SEG_ROWS
    nrows = jnp.where(wid == NW - 1, SEG_LAST, SEG_ROWS)

    pltpu.sync_copy(h_hbm.at[pl.ds(lo, SEG_LAST)], h_v)
    pltpu.sync_copy(batch_hbm.at[pl.ds(lo, SEG_LAST)], batch_v)
    pltpu.sync_copy(zero_hbm.at[pl.ds(0, G)], slab_v)

    iota16 = lax.iota(jnp.int32, _LANE16)

    def row(r, carry):
        g = batch_v[r]
        gidx = jnp.full((_LANE16,), g, jnp.int32)
        for k in range(H // _LANE16):
            cidx = iota16 + (k * _LANE16)
            vec = h_v[r, pl.ds(k * _LANE16, _LANE16)]
            cur = plsc.load_gather(slab_v, [gidx, cidx])
            plsc.store_scatter(slab_v, [gidx, cidx], jnp.maximum(cur, vec))
        return carry

    lax.fori_loop(0, nrows, row, 0)
    pltpu.sync_copy(slab_v, out_hbm.at[wid])


_segmax_call = pl.kernel(
    _segmax_body,
    out_type=jax.ShapeDtypeStruct((NW, G, H), jnp.float32),
    mesh=_mesh,
    scratch_types=[
        pltpu.VMEM((SEG_LAST, H), jnp.float32),
        pltpu.VMEM((SEG_LAST,), jnp.int32),
        pltpu.VMEM((G, H), jnp.float32),
    ],
)


def _t1_body(degp_ref, x_ref, w1_ref, y1_ref, dinv_ref):
    deg = 1.0 + degp_ref[0, :, 0:1] + degp_ref[1, :, 0:1]
    dinv = lax.rsqrt(deg)
    xw = jnp.dot(x_ref[:], w1_ref[:], preferred_element_type=jnp.float32)
    y1_ref[:] = xw * dinv
    dinv_ref[:] = dinv


def _t1(degp, x, W1):
    return pl.pallas_call(
        _t1_body,
        out_shape=(
            jax.ShapeDtypeStruct((N, H), jnp.float32),
            jax.ShapeDtypeStruct((N, 1), jnp.float32),
        ),
    )(degp, x, W1)


def _bn_relu(parts, y, dinv, b, g, be):
    acc = parts[0] + parts[1] + y
    t = acc * dinv + b
    mu = jnp.mean(t, axis=0, keepdims=True)
    var = jnp.mean((t - mu) * (t - mu), axis=0, keepdims=True)
    return jnp.maximum((t - mu) * lax.rsqrt(var + 1e-5) * g + be, 0.0)


def _tmid_body(p_ref, y_ref, dinv_ref, b_ref, g_ref, be_ref, w_ref, out_ref):
    hh = _bn_relu(p_ref[:], y_ref[:], dinv_ref[:], b_ref[:], g_ref[:], be_ref[:])
    out_ref[:] = jnp.dot(hh, w_ref[:], preferred_element_type=jnp.float32) * dinv_ref[:]


def _tmid(parts, y, dinv, b, g, be, Wn):
    return pl.pallas_call(
        _tmid_body,
        out_shape=jax.ShapeDtypeStruct((N, H), jnp.float32),
    )(parts, y, dinv, b, g, be, Wn)


def _tlast_body(p_ref, y_ref, dinv_ref, b_ref, g_ref, be_ref, out_ref):
    out_ref[:] = _bn_relu(p_ref[:], y_ref[:], dinv_ref[:], b_ref[:], g_ref[:],
                          be_ref[:])


def _tlast(parts, y, dinv, b, g, be):
    return pl.pallas_call(
        _tlast_body,
        out_shape=jax.ShapeDtypeStruct((N, H), jnp.float32),
    )(parts, y, dinv, b, g, be)


def _head_body(h_ref, batch_ref, slabs_ref, a_ref, bmid_ref, c_ref, lb1_ref,
               lw2_ref, lb2_ref, out_ref):
    ohT = (lax.broadcasted_iota(jnp.int32, (G, N), 0) == batch_ref[:]).astype(
        jnp.float32)
    ssum = jnp.dot(ohT, h_ref[:], preferred_element_type=jnp.float32)
    cnt = jnp.sum(ohT, axis=1, keepdims=True)
    mean = ssum / jnp.maximum(cnt, 1.0)
    mx = jnp.max(slabs_ref[:], axis=0)
    z = (jnp.dot(mean, a_ref[:], preferred_element_type=jnp.float32)
         + jnp.dot(mx, bmid_ref[:], preferred_element_type=jnp.float32)
         + jnp.dot(ssum, c_ref[:], preferred_element_type=jnp.float32)
         + lb1_ref[:])
    z = jnp.maximum(z, 0.0)
    out_ref[:] = jnp.dot(z, lw2_ref[:], preferred_element_type=jnp.float32) + lb2_ref[:]


def _head(h3, batchT, slabs, LW1, Lb1, LW2, Lb2):
    return pl.pallas_call(
        _head_body,
        out_shape=jax.ShapeDtypeStruct((G, C), jnp.float32),
    )(h3, batchT, slabs, LW1[0:H], LW1[H:2 * H], LW1[2 * H:3 * H],
      Lb1.reshape(1, H), LW2, Lb2.reshape(1, C))


def kernel(x, edge_index, batch, W1, b1, W2, b2, W3, b3,
           g1, be1, g2, be2, g3, be3, LW1, Lb1, LW2, Lb2):
    src2d = edge_index[0].reshape(E // K, K)
    dst2d = edge_index[1].reshape(E // K, K)
    zeros64 = jnp.zeros((N, H), jnp.float32)
    zeros16 = jnp.zeros((N, DW), jnp.float32)
    ones16 = jnp.ones((K, DW), jnp.float32)

    degp = _deg_call(dst2d, zeros16, ones16)
    y1, dinv = _t1(degp, x, W1)
    p1 = _scatter_call(y1, src2d, dst2d, zeros64)
    y2 = _tmid(p1, y1, dinv, b1.reshape(1, H), g1.reshape(1, H),
               be1.reshape(1, H), W2)
    p2 = _scatter_call(y2, src2d, dst2d, zeros64)
    y3 = _tmid(p2, y2, dinv, b2.reshape(1, H), g2.reshape(1, H),
               be2.reshape(1, H), W3)
    p3 = _scatter_call(y3, src2d, dst2d, zeros64)
    h3 = _tlast(p3, y3, dinv, b3.reshape(1, H), g3.reshape(1, H),
                be3.reshape(1, H))
    slabs = _segmax_call(h3, batch, zeros64)
    out = _head(h3, batch.reshape(1, N), slabs, LW1, Lb1, LW2, Lb2)
    return out


# speedup vs baseline: 18.4139x; 18.4139x over previous
import jax
import jax.numpy as jnp
from jax import lax
from jax.experimental import pallas as pl
from jax.experimental.pallas import tpu as pltpu
from jax.experimental.pallas import tpu_sc as plsc

N = 10000
E = 320000
F_IN = 128
H = 64
C = 1
G = 128

NC = 2
NS = 16
NW = NC * NS
K = 125
CH = E // NW // K
WPAD = 128

_mesh = plsc.VectorSubcoreMesh(core_axis_name="c", subcore_axis_name="s")


def _deg_body(dst_hbm, zero_hbm, ones_hbm, out_hbm, acc_sh, dst_v, ones_v):
    c = lax.axis_index("c")
    s = lax.axis_index("s")

    @pl.when(s == 0)
    def _():
        pltpu.sync_copy(zero_hbm, acc_sh)

    plsc.subcore_barrier()
    pltpu.sync_copy(ones_hbm, ones_v)
    row0 = c * (E // K // NC) + s * CH
    pltpu.sync_copy(dst_hbm.at[pl.ds(row0, CH)], dst_v)

    def chunk(i, carry):
        pltpu.sync_copy(ones_v, acc_sh.at[dst_v.at[i]], add=True)
        return carry

    lax.fori_loop(0, CH, chunk, 0)
    plsc.subcore_barrier()

    @pl.when(s == 0)
    def _():
        pltpu.sync_copy(acc_sh, out_hbm.at[c])


_deg_call = pl.kernel(
    _deg_body,
    out_type=jax.ShapeDtypeStruct((NC, N), jnp.float32),
    mesh=_mesh,
    scratch_types=[
        pltpu.VMEM_SHARED((N,), jnp.float32),
        pltpu.VMEM((CH, K), jnp.int32),
        pltpu.VMEM((K,), jnp.float32),
    ],
)


def _scatter_body(y_hbm, src_hbm, dst_hbm, zero_hbm, out_hbm,
                  acc_sh, src_v, dst_v, rows_v, sem):
    c = lax.axis_index("c")
    s = lax.axis_index("s")

    @pl.when(s == 0)
    def _():
        pltpu.sync_copy(zero_hbm, acc_sh)

    plsc.subcore_barrier()
    row0 = c * (E // K // NC) + s * CH
    pltpu.sync_copy(src_hbm.at[pl.ds(row0, CH)], src_v)
    pltpu.sync_copy(dst_hbm.at[pl.ds(row0, CH)], dst_v)

    def chunk(i, carry):
        pltpu.async_copy(y_hbm.at[src_v.at[i]], rows_v, sem).wait()
        pltpu.sync_copy(rows_v, acc_sh.at[dst_v.at[i]], add=True)
        return carry

    lax.fori_loop(0, CH, chunk, 0)
    plsc.subcore_barrier()

    @pl.when(s == 0)
    def _():
        pltpu.sync_copy(acc_sh, out_hbm.at[c])


_scatter_call = pl.kernel(
    _scatter_body,
    out_type=jax.ShapeDtypeStruct((NC, N, WPAD), jnp.float32),
    mesh=_mesh,
    scratch_types=[
        pltpu.VMEM_SHARED((N, WPAD), jnp.float32),
        pltpu.VMEM((CH, K), jnp.int32),
        pltpu.VMEM((CH, K), jnp.int32),
        pltpu.VMEM((K, WPAD), jnp.float32),
        pltpu.SemaphoreType.DMA,
    ],
)


def _pad128(v):
    return jnp.concatenate([v, jnp.zeros((N, WPAD - H), jnp.float32)], axis=1)


def _t1_body(degp_ref, x_ref, w1_ref, y1_ref, dinv_ref):
    deg = 1.0 + degp_ref[0] + degp_ref[1]
    dinv = lax.rsqrt(deg)
    xw = jnp.dot(x_ref[:], w1_ref[:], preferred_element_type=jnp.float32)
    y1_ref[:] = _pad128(xw * dinv)
    dinv_ref[:] = dinv


def _t1(degp, x, W1):
    return pl.pallas_call(
        _t1_body,
        out_shape=(
            jax.ShapeDtypeStruct((N, WPAD), jnp.float32),
            jax.ShapeDtypeStruct((N, 1), jnp.float32),
        ),
    )(degp, x, W1)


def _bn_relu(parts, y, dinv, b, g, be):
    acc = parts[0, :, :H] + parts[1, :, :H] + y[:, :H]
    t = acc * dinv + b
    mu = jnp.mean(t, axis=0, keepdims=True)
    var = jnp.mean((t - mu) * (t - mu), axis=0, keepdims=True)
    return jnp.maximum((t - mu) * lax.rsqrt(var + 1e-5) * g + be, 0.0)


def _tmid_body(p_ref, y_ref, dinv_ref, b_ref, g_ref, be_ref, w_ref, out_ref):
    hh = _bn_relu(p_ref[:], y_ref[:], dinv_ref[:], b_ref[:], g_ref[:], be_ref[:])
    yn = jnp.dot(hh, w_ref[:], preferred_element_type=jnp.float32) * dinv_ref[:]
    out_ref[:] = _pad128(yn)


def _tmid(parts, y, dinv, b, g, be, Wn):
    return pl.pallas_call(
        _tmid_body,
        out_shape=jax.ShapeDtypeStruct((N, WPAD), jnp.float32),
    )(parts, y, dinv, b.reshape(1, H), g.reshape(1, H), be.reshape(1, H), Wn)


def _tlast_body(p_ref, y_ref, dinv_ref, b_ref, g_ref, be_ref, out_ref):
    out_ref[:] = _bn_relu(p_ref[:], y_ref[:], dinv_ref[:], b_ref[:], g_ref[:],
                          be_ref[:])


def _tlast(parts, y, dinv, b, g, be):
    return pl.pallas_call(
        _tlast_body,
        out_shape=jax.ShapeDtypeStruct((N, H), jnp.float32),
    )(parts, y, dinv, b.reshape(1, H), g.reshape(1, H), be.reshape(1, H))


def _head_body(h_ref, batchT_ref, batch2_ref, a_ref, bmid_ref, c_ref, lb1_ref,
               lw2_ref, lb2_ref, out_ref):
    hh = h_ref[:]
    ohT = (lax.broadcasted_iota(jnp.int32, (G, N), 0) == batchT_ref[:]).astype(
        jnp.float32)
    ssum = jnp.dot(ohT, hh, preferred_element_type=jnp.float32)
    cnt = jnp.sum(ohT, axis=1, keepdims=True)
    mean = ssum / jnp.maximum(cnt, 1.0)

    bm = batch2_ref[:]
    m = hh
    k = 1
    while k < N:
        bsh = jnp.concatenate(
            [jnp.full((k, 1), -1, jnp.int32), bm[:-k]], axis=0)
        sh = jnp.concatenate([jnp.zeros((k, H), jnp.float32), m[:-k]], axis=0)
        m = jnp.where(bm == bsh, jnp.maximum(m, sh), m)
        k *= 2
    bnext = jnp.concatenate(
        [bm[1:], jnp.full((1, 1), -1, jnp.int32)], axis=0)
    mx = jnp.dot(ohT, jnp.where(bm != bnext, m, 0.0),
                 preferred_element_type=jnp.float32)

    z = (jnp.dot(mean, a_ref[:], preferred_element_type=jnp.float32)
         + jnp.dot(mx, bmid_ref[:], preferred_element_type=jnp.float32)
         + jnp.dot(ssum, c_ref[:], preferred_element_type=jnp.float32)
         + lb1_ref[:])
    z = jnp.maximum(z, 0.0)
    out_ref[:] = jnp.dot(z, lw2_ref[:], preferred_element_type=jnp.float32) + lb2_ref[:]


def _head(h3, batch, LW1, Lb1, LW2, Lb2):
    return pl.pallas_call(
        _head_body,
        out_shape=jax.ShapeDtypeStruct((G, C), jnp.float32),
    )(h3, batch.reshape(1, N), batch.reshape(N, 1),
      LW1[0:H], LW1[H:2 * H], LW1[2 * H:3 * H],
      Lb1.reshape(1, H), LW2, Lb2.reshape(1, C))


def kernel(x, edge_index, batch, W1, b1, W2, b2, W3, b3,
           g1, be1, g2, be2, g3, be3, LW1, Lb1, LW2, Lb2):
    src2d = edge_index[0].reshape(E // K, K)
    dst2d = edge_index[1].reshape(E // K, K)
    zerosP = jnp.zeros((N, WPAD), jnp.float32)
    zeros1 = jnp.zeros((N,), jnp.float32)
    ones1 = jnp.ones((K,), jnp.float32)

    degp = _deg_call(dst2d, zeros1, ones1)
    y1, dinv = _t1(degp.reshape(NC, N, 1), x, W1)
    p1 = _scatter_call(y1, src2d, dst2d, zerosP)
    y2 = _tmid(p1, y1, dinv, b1, g1, be1, W2)
    p2 = _scatter_call(y2, src2d, dst2d, zerosP)
    y3 = _tmid(p2, y2, dinv, b2, g2, be2, W3)
    p3 = _scatter_call(y3, src2d, dst2d, zerosP)
    h3 = _tlast(p3, y3, dinv, b3, g3, be3)
    out = _head(h3, batch, LW1, Lb1, LW2, Lb2)
    return out


# speedup vs baseline: 24.0824x; 1.3078x over previous
import jax
import jax.numpy as jnp
from jax import lax
from jax.experimental import pallas as pl
from jax.experimental.pallas import tpu as pltpu
from jax.experimental.pallas import tpu_sc as plsc

N = 10000
E = 320000
F_IN = 128
H = 64
C = 1
G = 128

NC = 2
NS = 16
NW = NC * NS
K = 125
CH = E // NW // K
WPAD = 128

_mesh = plsc.VectorSubcoreMesh(core_axis_name="c", subcore_axis_name="s")


def _deg_body(dst_hbm, zero_hbm, ones_hbm, out_hbm, acc_sh, dst_v, ones_v):
    c = lax.axis_index("c")
    s = lax.axis_index("s")

    @pl.when(s == 0)
    def _():
        pltpu.sync_copy(zero_hbm, acc_sh)

    plsc.subcore_barrier()
    pltpu.sync_copy(ones_hbm, ones_v)
    row0 = c * (E // K // NC) + s * CH
    pltpu.sync_copy(dst_hbm.at[pl.ds(row0, CH)], dst_v)

    def chunk(i, carry):
        pltpu.sync_copy(ones_v, acc_sh.at[dst_v.at[i]], add=True)
        return carry

    lax.fori_loop(0, CH, chunk, 0)
    plsc.subcore_barrier()

    @pl.when(s == 0)
    def _():
        pltpu.sync_copy(acc_sh, out_hbm.at[c])


_deg_call = pl.kernel(
    _deg_body,
    out_type=jax.ShapeDtypeStruct((NC, N), jnp.float32),
    mesh=_mesh,
    scratch_types=[
        pltpu.VMEM_SHARED((N,), jnp.float32),
        pltpu.VMEM((CH, K), jnp.int32),
        pltpu.VMEM((K,), jnp.float32),
    ],
)


HCH = CH // 2


def _scatter_body(y_hbm, src_hbm, dst_hbm, zero_hbm, out_hbm,
                  acc_sh, src_v, dst_v, rows0, rows1, semA, semB):
    c = lax.axis_index("c")
    s = lax.axis_index("s")

    @pl.when(s == 0)
    def _():
        pltpu.sync_copy(zero_hbm, acc_sh)

    row0 = c * (E // K // NC) + s * CH
    plsc.subcore_barrier()

    for p in range(2):
        pltpu.sync_copy(src_hbm.at[pl.ds(row0 + p * HCH, HCH)], src_v)
        pltpu.sync_copy(dst_hbm.at[pl.ds(row0 + p * HCH, HCH)], dst_v)
        pltpu.async_copy(y_hbm.at[src_v.at[0]], rows0, semA)

        def chunk2(j, carry):
            i0 = 2 * j
            pltpu.make_async_copy(y_hbm.at[src_v.at[i0]], rows0, semA).wait()
            pltpu.async_copy(y_hbm.at[src_v.at[i0 + 1]], rows1, semB)
            pltpu.sync_copy(rows0, acc_sh.at[dst_v.at[i0]], add=True)
            pltpu.make_async_copy(y_hbm.at[src_v.at[i0]], rows1, semB).wait()

            @pl.when(j + 1 < HCH // 2)
            def _():
                pltpu.async_copy(y_hbm.at[src_v.at[i0 + 2]], rows0, semA)

            pltpu.sync_copy(rows1, acc_sh.at[dst_v.at[i0 + 1]], add=True)
            return carry

        lax.fori_loop(0, HCH // 2, chunk2, 0)
    plsc.subcore_barrier()

    @pl.when(s == 0)
    def _():
        pltpu.sync_copy(acc_sh, out_hbm.at[c])


_scatter_call = pl.kernel(
    _scatter_body,
    out_type=jax.ShapeDtypeStruct((NC, N, WPAD), jnp.float32),
    mesh=_mesh,
    scratch_types=[
        pltpu.VMEM_SHARED((N, WPAD), jnp.float32),
        pltpu.VMEM((HCH, K), jnp.int32),
        pltpu.VMEM((HCH, K), jnp.int32),
        pltpu.VMEM((K, WPAD), jnp.float32),
        pltpu.VMEM((K, WPAD), jnp.float32),
        pltpu.SemaphoreType.DMA,
        pltpu.SemaphoreType.DMA,
    ],
)


def _pad128(v):
    return jnp.concatenate([v, jnp.zeros((N, WPAD - H), jnp.float32)], axis=1)


def _t1_body(degp_ref, x_ref, w1_ref, y1_ref, dinv_ref):
    deg = 1.0 + degp_ref[0] + degp_ref[1]
    dinv = lax.rsqrt(deg)
    xw = jnp.dot(x_ref[:], w1_ref[:], preferred_element_type=jnp.float32)
    y1_ref[:] = _pad128(xw * dinv)
    dinv_ref[:] = dinv


def _t1(degp, x, W1):
    return pl.pallas_call(
        _t1_body,
        out_shape=(
            jax.ShapeDtypeStruct((N, WPAD), jnp.float32),
            jax.ShapeDtypeStruct((N, 1), jnp.float32),
        ),
    )(degp, x, W1)


def _bn_relu(parts, y, dinv, b, g, be):
    acc = parts[0, :, :H] + parts[1, :, :H] + y[:, :H]
    t = acc * dinv + b
    mu = jnp.mean(t, axis=0, keepdims=True)
    var = jnp.mean((t - mu) * (t - mu), axis=0, keepdims=True)
    return jnp.maximum((t - mu) * lax.rsqrt(var + 1e-5) * g + be, 0.0)


def _tmid_body(p_ref, y_ref, dinv_ref, b_ref, g_ref, be_ref, w_ref, out_ref):
    hh = _bn_relu(p_ref[:], y_ref[:], dinv_ref[:], b_ref[:], g_ref[:], be_ref[:])
    yn = jnp.dot(hh, w_ref[:], preferred_element_type=jnp.float32) * dinv_ref[:]
    out_ref[:] = _pad128(yn)


def _tmid(parts, y, dinv, b, g, be, Wn):
    return pl.pallas_call(
        _tmid_body,
        out_shape=jax.ShapeDtypeStruct((N, WPAD), jnp.float32),
    )(parts, y, dinv, b.reshape(1, H), g.reshape(1, H), be.reshape(1, H), Wn)


def _tlast_body(p_ref, y_ref, dinv_ref, b_ref, g_ref, be_ref, out_ref):
    out_ref[:] = _bn_relu(p_ref[:], y_ref[:], dinv_ref[:], b_ref[:], g_ref[:],
                          be_ref[:])


def _tlast(parts, y, dinv, b, g, be):
    return pl.pallas_call(
        _tlast_body,
        out_shape=jax.ShapeDtypeStruct((N, H), jnp.float32),
    )(parts, y, dinv, b.reshape(1, H), g.reshape(1, H), be.reshape(1, H))


def _head_body(h_ref, batchT_ref, batch2_ref, a_ref, bmid_ref, c_ref, lb1_ref,
               lw2_ref, lb2_ref, out_ref):
    hh = h_ref[:]
    ohT = (lax.broadcasted_iota(jnp.int32, (G, N), 0) == batchT_ref[:]).astype(
        jnp.float32)
    ssum = jnp.dot(ohT, hh, preferred_element_type=jnp.float32)
    cnt = jnp.sum(ohT, axis=1, keepdims=True)
    mean = ssum / jnp.maximum(cnt, 1.0)

    bm = batch2_ref[:]
    m = hh
    k = 1
    while k < N:
        bsh = jnp.concatenate(
            [jnp.full((k, 1), -1, jnp.int32), bm[:-k]], axis=0)
        sh = jnp.concatenate([jnp.zeros((k, H), jnp.float32), m[:-k]], axis=0)
        m = jnp.where(bm == bsh, jnp.maximum(m, sh), m)
        k *= 2
    bnext = jnp.concatenate(
        [bm[1:], jnp.full((1, 1), -1, jnp.int32)], axis=0)
    mx = jnp.dot(ohT, jnp.where(bm != bnext, m, 0.0),
                 preferred_element_type=jnp.float32)

    z = (jnp.dot(mean, a_ref[:], preferred_element_type=jnp.float32)
         + jnp.dot(mx, bmid_ref[:], preferred_element_type=jnp.float32)
         + jnp.dot(ssum, c_ref[:], preferred_element_type=jnp.float32)
         + lb1_ref[:])
    z = jnp.maximum(z, 0.0)
    out_ref[:] = jnp.dot(z, lw2_ref[:], preferred_element_type=jnp.float32) + lb2_ref[:]


def _head(h3, batch, LW1, Lb1, LW2, Lb2):
    return pl.pallas_call(
        _head_body,
        out_shape=jax.ShapeDtypeStruct((G, C), jnp.float32),
    )(h3, batch.reshape(1, N), batch.reshape(N, 1),
      LW1[0:H], LW1[H:2 * H], LW1[2 * H:3 * H],
      Lb1.reshape(1, H), LW2, Lb2.reshape(1, C))


def kernel(x, edge_index, batch, W1, b1, W2, b2, W3, b3,
           g1, be1, g2, be2, g3, be3, LW1, Lb1, LW2, Lb2):
    src2d = edge_index[0].reshape(E // K, K)
    dst2d = edge_index[1].reshape(E // K, K)
    zerosP = jnp.zeros((N, WPAD), jnp.float32)
    zeros1 = jnp.zeros((N,), jnp.float32)
    ones1 = jnp.ones((K,), jnp.float32)

    degp = _deg_call(dst2d, zeros1, ones1)
    y1, dinv = _t1(degp.reshape(NC, N, 1), x, W1)
    p1 = _scatter_call(y1, src2d, dst2d, zerosP)
    y2 = _tmid(p1, y1, dinv, b1, g1, be1, W2)
    p2 = _scatter_call(y2, src2d, dst2d, zerosP)
    y3 = _tmid(p2, y2, dinv, b2, g2, be2, W3)
    p3 = _scatter_call(y3, src2d, dst2d, zerosP)
    h3 = _tlast(p3, y3, dinv, b3, g3, be3)
    out = _head(h3, batch, LW1, Lb1, LW2, Lb2)
    return out
